# baseline (speedup 1.0000x reference)
import functools

import jax
import jax.numpy as jnp
from jax import lax
from jax.experimental import pallas as pl
from jax.experimental.pallas import tpu as pltpu
from jax.experimental.pallas import tpu_sc as plsc

_V = 1000
_B = 1024
_S = 50
_NW = 32
_RPW = _B // _NW


def _sc_onehot(x_hbm, z_hbm, out_hbm, idx_v, slab0, slab1, s0, s1):
    sid = lax.axis_index("s")
    cid = lax.axis_index("c")
    wid = sid * 2 + cid
    row0 = wid * _RPW

    pltpu.sync_copy(x_hbm.at[pl.ds(row0 * _S, _RPW * _S)], idx_v)
    pltpu.sync_copy(z_hbm, slab0)
    pltpu.sync_copy(z_hbm, slab1)

    ones16 = jnp.ones((16,), jnp.float32)
    zeros16 = jnp.zeros((16,), jnp.float32)
    z16 = jnp.zeros((16,), jnp.int32)
    iota16 = lax.iota(jnp.int32, 16)
    ntv = (_S + 15) // 16

    def put(slab, r, val16):
        for v in range(ntv):
            tok = iota16 + v * 16
            mask = tok < _S
            ids = idx_v[pl.ds(r * _S + v * 16, 16)] if v < 3 else (
                idx_v[pl.ds(r * _S + _S - 16, 16)]
            )
            if v == 3:
                tok = iota16 + (_S - 16)
                mask = iota16 >= (16 - (_S - 48))
            plsc.store_scatter(slab, [z16, tok, ids], val16, mask=mask)

    slabs = (slab0, slab1)
    sems = (s0, s1)
    handles = [None, None]
    for r in range(_RPW):
        b = r % 2
        slab = slabs[b]
        if handles[b] is not None:
            handles[b].wait()
            put(slab, r - 2, zeros16)
        put(slab, r, ones16)
        handles[b] = pltpu.async_copy(
            slab, out_hbm.at[pl.ds(row0 + r, 1)], sems[b]
        )
    handles[0].wait()
    handles[1].wait()


def kernel(x):
    xi = x.reshape(-1).astype(jnp.int32)
    z = jnp.zeros((1, _S, _V), jnp.float32)
    mesh = plsc.VectorSubcoreMesh(core_axis_name="c", subcore_axis_name="s")
    run = functools.partial(
        pl.kernel,
        mesh=mesh,
        out_type=jax.ShapeDtypeStruct((_B, _S, _V), jnp.float32),
        scratch_types=[
            pltpu.VMEM((_RPW * _S,), jnp.int32),
            pltpu.VMEM((1, _S, _V), jnp.float32),
            pltpu.VMEM((1, _S, _V), jnp.float32),
            pltpu.SemaphoreType.DMA,
            pltpu.SemaphoreType.DMA,
        ],
        compiler_params=pltpu.CompilerParams(
            needs_layout_passes=False, use_tc_tiling_on_sc=True
        ),
    )(_sc_onehot)
    return run(xi, z)


# speedup vs baseline: 1.0017x; 1.0017x over previous
import functools

import jax
import jax.numpy as jnp
from jax import lax
from jax.experimental import pallas as pl
from jax.experimental.pallas import tpu as pltpu
from jax.experimental.pallas import tpu_sc as plsc

_V = 1000
_B = 1024
_S = 50
_NW = 32
_RPW = _B // _NW


def _sc_onehot(x_hbm, z_hbm, out_hbm, idx_v, slab0, slab1, s0, s1):
    sid = lax.axis_index("s")
    cid = lax.axis_index("c")
    wid = sid * 2 + cid
    row0 = wid * _RPW

    pltpu.sync_copy(x_hbm.at[pl.ds(row0 * _S, _RPW * _S)], idx_v)
    pltpu.sync_copy(z_hbm, slab0)
    pltpu.sync_copy(z_hbm, slab1)

    ones16 = jnp.ones((16,), jnp.float32)
    zeros16 = jnp.zeros((16,), jnp.float32)
    z16 = jnp.zeros((16,), jnp.int32)
    iota16 = lax.iota(jnp.int32, 16)
    ntv = (_S + 15) // 16

    def put(slab, r, val16):
        for v in range(ntv):
            tok = iota16 + v * 16
            mask = tok < _S
            ids = idx_v[pl.ds(r * _S + v * 16, 16)] if v < 3 else (
                idx_v[pl.ds(r * _S + _S - 16, 16)]
            )
            if v == 3:
                tok = iota16 + (_S - 16)
                mask = iota16 >= (16 - (_S - 48))
            plsc.store_scatter(slab, [z16, tok, ids], val16, mask=mask)

    slabs = (slab0, slab1)
    sems = (s0, s1)
    handles = [None, None]
    for r in range(_RPW):
        b = r % 2
        slab = slabs[b]
        if handles[b] is not None:
            handles[b].wait()
            put(slab, r - 2, zeros16)
        put(slab, r, ones16)
        handles[b] = pltpu.async_copy(
            slab, out_hbm.at[pl.ds(row0 + r, 1)], sems[b]
        )
    handles[0].wait()
    handles[1].wait()


def kernel(x):
    xi = x.reshape(-1).astype(jnp.int32)
    z = jnp.zeros((1, _S, _V), jnp.float32)
    mesh = plsc.VectorSubcoreMesh(core_axis_name="c", subcore_axis_name="s")
    run = functools.partial(
        pl.kernel,
        mesh=mesh,
        out_type=jax.ShapeDtypeStruct((_B, _S, _V), jnp.float32),
        scratch_types=[
            pltpu.VMEM((_RPW * _S,), jnp.int32),
            pltpu.VMEM((1, _S, _V), jnp.float32),
            pltpu.VMEM((1, _S, _V), jnp.float32),
            pltpu.SemaphoreType.DMA,
            pltpu.SemaphoreType.DMA,
        ],
        compiler_params=pltpu.CompilerParams(
            needs_layout_passes=False, use_tc_tiling_on_sc=True
        ),
    )(_sc_onehot)
    out = run(xi, z)
    return pl.pallas_call(
        lambda i_ref, o_ref: None,
        in_specs=[pl.BlockSpec(memory_space=pl.ANY)],
        out_specs=pl.BlockSpec(memory_space=pl.ANY),
        out_shape=jax.ShapeDtypeStruct((_B, _S, _V), jnp.float32),
        input_output_aliases={0: 0},
    )(out)
